# Initial kernel scaffold; baseline (speedup 1.0000x reference)
import functools

import jax
import jax.numpy as jnp
from jax import lax
from jax.experimental import pallas as pl
from jax.experimental.pallas import tpu as pltpu
from jax.experimental.pallas import tpu_sc as plsc

N = 10000
E = 320000
HF = 200
HP = 208
ZP = 16
NC = 2
NS = 16
EB = 80

NB1 = E // NS // EB
NB2 = E // (NC * NS) // EB
ROWS_PER_TILE = N // NS
ZCHUNK = 125
LANES = 16


def _mm_body(x_ref, w1_ref, w2_ref, o1_ref, o2_ref):
    xa = x_ref[:, :768]
    xb = x_ref[:, 768:]
    o1_ref[...] = jnp.dot(xa, w1_ref[...], preferred_element_type=jnp.float32)
    o2_ref[...] = jnp.dot(xb, w2_ref[...], preferred_element_type=jnp.float32)


def _matmuls(x, w1p, w2p):
    bm = 1000
    return pl.pallas_call(
        _mm_body,
        grid=(N // bm,),
        in_specs=[
            pl.BlockSpec((bm, 1536), lambda i: (i, 0)),
            pl.BlockSpec((768, HP), lambda i: (0, 0)),
            pl.BlockSpec((768, HP), lambda i: (0, 0)),
        ],
        out_specs=[
            pl.BlockSpec((bm, HP), lambda i: (i, 0)),
            pl.BlockSpec((bm, HP), lambda i: (i, 0)),
        ],
        out_shape=[
            jax.ShapeDtypeStruct((N, HP), jnp.float32),
            jax.ShapeDtypeStruct((N, HP), jnp.float32),
        ],
    )(x, w1p, w2p)


def _sc_prop(table, out, srcv, dstv, ewv, rows, acc, s, nb, width):
    nvec = width // LANES

    def batch(j, carry):
        pltpu.sync_copy(table.at[srcv.at[j]], rows)

        def scale_row(e, c2):
            w = ewv[j, e]
            for k in range(nvec):
                rows[e, pl.ds(k * LANES, LANES)] = (
                    rows[e, pl.ds(k * LANES, LANES)] * w
                )
            return c2

        lax.fori_loop(0, EB, scale_row, 0)

        pltpu.sync_copy(rows, acc.at[dstv.at[j]], add=True)
        return carry

    lax.fori_loop(0, nb, batch, 0)
    plsc.subcore_barrier()

    for r in range(ROWS_PER_TILE // ZCHUNK):
        base = s * ROWS_PER_TILE + r * ZCHUNK
        pltpu.sync_copy(acc.at[pl.ds(base, ZCHUNK)], out.at[pl.ds(base, ZCHUNK)])


def _prop1(h1, h2, srcr, dstr, ewr, zeros):
    mesh = plsc.VectorSubcoreMesh(
        core_axis_name="c", subcore_axis_name="s", num_cores=NC, num_subcores=NS
    )

    @functools.partial(
        pl.kernel,
        out_type=[
            jax.ShapeDtypeStruct((N, HP), jnp.float32),
            jax.ShapeDtypeStruct((N, HP), jnp.float32),
        ],
        mesh=mesh,
        scratch_types=[
            pltpu.VMEM_SHARED((N, HP), jnp.float32),
            pltpu.VMEM((NB1, EB), jnp.int32),
            pltpu.VMEM((NB1, EB), jnp.int32),
            pltpu.VMEM((NB1, EB), jnp.float32),
            pltpu.VMEM((EB, HP), jnp.float32),
        ],
    )
    def k(h1_hbm, h2_hbm, src_hbm, dst_hbm, ew_hbm, z_hbm, o1, o2,
          acc, srcv, dstv, ewv, rows):
        c = lax.axis_index("c")
        s = lax.axis_index("s")
        pltpu.sync_copy(src_hbm.at[s], srcv)
        pltpu.sync_copy(dst_hbm.at[s], dstv)
        pltpu.sync_copy(ew_hbm.at[s], ewv)
        for r in range(ROWS_PER_TILE // ZCHUNK):
            base = s * ROWS_PER_TILE + r * ZCHUNK
            pltpu.sync_copy(z_hbm, acc.at[pl.ds(base, ZCHUNK)])
        plsc.subcore_barrier()

        @pl.when(c == 0)
        def _():
            _sc_prop(h1_hbm, o1, srcv, dstv, ewv, rows, acc, s, NB1, HP)

        @pl.when(c == 1)
        def _():
            _sc_prop(h2_hbm, o2, srcv, dstv, ewv, rows, acc, s, NB1, HP)

    return k(h1, h2, srcr, dstr, ewr, zeros)


def _prop2(zp, srcr, dstr, ewr, zeros):
    mesh = plsc.VectorSubcoreMesh(
        core_axis_name="c", subcore_axis_name="s", num_cores=NC, num_subcores=NS
    )

    @functools.partial(
        pl.kernel,
        out_type=[
            jax.ShapeDtypeStruct((N, ZP), jnp.float32),
            jax.ShapeDtypeStruct((N, ZP), jnp.float32),
        ],
        mesh=mesh,
        scratch_types=[
            pltpu.VMEM_SHARED((N, ZP), jnp.float32),
            pltpu.VMEM((NB2, EB), jnp.int32),
            pltpu.VMEM((NB2, EB), jnp.int32),
            pltpu.VMEM((NB2, EB), jnp.float32),
            pltpu.VMEM((EB, ZP), jnp.float32),
        ],
    )
    def k(zp_hbm, src_hbm, dst_hbm, ew_hbm, z_hbm, o1, o2,
          acc, srcv, dstv, ewv, rows):
        c = lax.axis_index("c")
        s = lax.axis_index("s")
        w = s * NC + c
        pltpu.sync_copy(src_hbm.at[w], srcv)
        pltpu.sync_copy(dst_hbm.at[w], dstv)
        pltpu.sync_copy(ew_hbm.at[w], ewv)
        for r in range(ROWS_PER_TILE // ZCHUNK):
            base = s * ROWS_PER_TILE + r * ZCHUNK
            pltpu.sync_copy(z_hbm, acc.at[pl.ds(base, ZCHUNK)])
        plsc.subcore_barrier()

        @pl.when(c == 0)
        def _():
            _sc_prop(zp_hbm, o1, srcv, dstv, ewv, rows, acc, s, NB2, ZP)

        @pl.when(c == 1)
        def _():
            _sc_prop(zp_hbm, o2, srcv, dstv, ewv, rows, acc, s, NB2, ZP)

    return k(zp, srcr, dstr, ewr, zeros)


def _mid_body(a1_ref, a2_ref, b1_ref, b2_ref, wc1_ref, wc2_ref, o_ref):
    o1 = jax.nn.relu(a1_ref[...] + b1_ref[...])
    o2 = jax.nn.relu(a2_ref[...] + b2_ref[...])
    z = jnp.dot(o1, wc1_ref[...], preferred_element_type=jnp.float32)
    z = z + jnp.dot(o2, wc2_ref[...], preferred_element_type=jnp.float32)
    o_ref[...] = z


def _mid(agg1, agg2, b1p_1, b1p_2, wc1p, wc2p):
    bm = 1000
    return pl.pallas_call(
        _mid_body,
        grid=(N // bm,),
        in_specs=[
            pl.BlockSpec((bm, HP), lambda i: (i, 0)),
            pl.BlockSpec((bm, HP), lambda i: (i, 0)),
            pl.BlockSpec((1, HP), lambda i: (0, 0)),
            pl.BlockSpec((1, HP), lambda i: (0, 0)),
            pl.BlockSpec((HP, ZP), lambda i: (0, 0)),
            pl.BlockSpec((HP, ZP), lambda i: (0, 0)),
        ],
        out_specs=pl.BlockSpec((bm, ZP), lambda i: (i, 0)),
        out_shape=jax.ShapeDtypeStruct((N, ZP), jnp.float32),
    )(agg1, agg2, b1p_1, b1p_2, wc1p, wc2p)


def _final_body(q0_ref, q1_ref, bc_ref, o_ref):
    t = q0_ref[...] + q1_ref[...]
    a0 = t[:, 0:1] + bc_ref[0, 0]
    a1 = t[:, 1:2] + bc_ref[0, 1]
    b0 = t[:, 2:3] + bc_ref[0, 2]
    b1 = t[:, 3:4] + bc_ref[0, 3]
    m1 = jnp.maximum(a0, a1)
    e0 = jnp.exp(a0 - m1)
    e1 = jnp.exp(a1 - m1)
    p10 = e0 / (e0 + e1)
    p11 = e1 / (e0 + e1)
    m2 = jnp.maximum(b0, b1)
    f0 = jnp.exp(b0 - m2)
    f1 = jnp.exp(b1 - m2)
    p20 = f0 / (f0 + f1)
    p21 = f1 / (f0 + f1)
    v0 = jnp.maximum(p10, p20)
    v1 = jnp.maximum(p11, p21)
    tot = v0 + v1
    o_ref[...] = jnp.concatenate([v0 / tot, v1 / tot], axis=1)


def _final(q0, q1, bc):
    bm = 1000
    return pl.pallas_call(
        _final_body,
        grid=(N // bm,),
        in_specs=[
            pl.BlockSpec((bm, ZP), lambda i: (i, 0)),
            pl.BlockSpec((bm, ZP), lambda i: (i, 0)),
            pl.BlockSpec((1, 4), lambda i: (0, 0)),
        ],
        out_specs=pl.BlockSpec((bm, 2), lambda i: (i, 0)),
        out_shape=jax.ShapeDtypeStruct((N, 2), jnp.float32),
    )(q0, q1, bc)


def kernel(x, edge_index, edge_weight, W1_e1, b1_e1, Wc_e1, bc_e1,
           W1_e2, b1_e2, Wc_e2, bc_e2):
    src = edge_index[0].astype(jnp.int32)
    dst = edge_index[1].astype(jnp.int32)
    ew = edge_weight.astype(jnp.float32)

    w1p_1 = jnp.pad(W1_e1, ((0, 0), (0, HP - HF)))
    w1p_2 = jnp.pad(W1_e2, ((0, 0), (0, HP - HF)))
    b1p_1 = jnp.pad(b1_e1, (0, HP - HF)).reshape(1, HP)
    b1p_2 = jnp.pad(b1_e2, (0, HP - HF)).reshape(1, HP)
    wc1p = jnp.zeros((HP, ZP), jnp.float32).at[:HF, 0:2].set(Wc_e1)
    wc2p = jnp.zeros((HP, ZP), jnp.float32).at[:HF, 2:4].set(Wc_e2)
    bc = jnp.concatenate([bc_e1, bc_e2]).reshape(1, 4)

    src1 = src.reshape(NS, NB1, EB)
    dst1 = dst.reshape(NS, NB1, EB)
    ew1 = ew.reshape(NS, NB1, EB)
    src2 = src.reshape(NC * NS, NB2, EB)
    dst2 = dst.reshape(NC * NS, NB2, EB)
    ew2 = ew.reshape(NC * NS, NB2, EB)
    zeros1 = jnp.zeros((ZCHUNK, HP), jnp.float32)
    zeros2 = jnp.zeros((ZCHUNK, ZP), jnp.float32)

    h1, h2 = _matmuls(x, w1p_1, w1p_2)
    agg1, agg2 = _prop1(h1, h2, src1, dst1, ew1, zeros1)
    zpk = _mid(agg1, agg2, b1p_1, b1p_2, wc1p, wc2p)
    q0, q1 = _prop2(zpk, src2, dst2, ew2, zeros2)
    return _final(q0, q1, bc)


# speedup vs baseline: 5.1611x; 5.1611x over previous
import functools

import jax
import jax.numpy as jnp
from jax import lax
from jax.experimental import pallas as pl
from jax.experimental.pallas import tpu as pltpu
from jax.experimental.pallas import tpu_sc as plsc

N = 10000
E = 320000
HF = 200
FC = 112
ZP = 16
NC = 2
NS = 16
EB = 80
SB = 25
NSB = E // NS // (SB * EB)
NB2 = E // (NC * NS) // EB
WCHUNK = 1000
NWTILES = N // WCHUNK
LANES = 16


def _mm_body(x_ref, w1a, w1b, w2a, w2b, o1a, o1b, o2a, o2b):
    xa = x_ref[:, :768]
    xb = x_ref[:, 768:]
    o1a[...] = jnp.dot(xa, w1a[...], preferred_element_type=jnp.float32)
    o1b[...] = jnp.dot(xa, w1b[...], preferred_element_type=jnp.float32)
    o2a[...] = jnp.dot(xb, w2a[...], preferred_element_type=jnp.float32)
    o2b[...] = jnp.dot(xb, w2b[...], preferred_element_type=jnp.float32)


def _matmuls(x, w1a, w1b, w2a, w2b):
    bm = 1000
    wspec = pl.BlockSpec((768, FC), lambda i: (0, 0))
    ospec = pl.BlockSpec((bm, FC), lambda i: (i, 0))
    osh = jax.ShapeDtypeStruct((N, FC), jnp.float32)
    return pl.pallas_call(
        _mm_body,
        grid=(N // bm,),
        in_specs=[pl.BlockSpec((bm, 1536), lambda i: (i, 0))] + [wspec] * 4,
        out_specs=[ospec] * 4,
        out_shape=[osh] * 4,
    )(x, w1a, w1b, w2a, w2b)


def _zero_own(z_hbm, acc, s):
    @pl.when(s < NWTILES)
    def _():
        base = pl.multiple_of(s * WCHUNK, 8)
        pltpu.sync_copy(z_hbm, acc.at[pl.ds(base, WCHUNK)])


def _writeout(acc, out, s):
    @pl.when(s < NWTILES)
    def _():
        base = pl.multiple_of(s * WCHUNK, 8)
        pltpu.sync_copy(acc.at[pl.ds(base, WCHUNK)], out.at[pl.ds(base, WCHUNK)])


def _scale_rows(rows, ewv, j, nvec):
    def scale_group(g, c2):
        wv = ewv[j, pl.ds(g * LANES, LANES)]
        for l in range(LANES):
            w = wv[l]
            for k in range(nvec):
                rows[g * LANES + l, pl.ds(k * LANES, LANES)] = (
                    rows[g * LANES + l, pl.ds(k * LANES, LANES)] * w
                )
        return c2

    lax.fori_loop(0, EB // LANES, scale_group, 0)


def _prop1(h1a, h1b, h2a, h2b, srcr, dstr, ewr, zeros):
    mesh = plsc.VectorSubcoreMesh(
        core_axis_name="c", subcore_axis_name="s", num_cores=NC, num_subcores=NS
    )
    osh = jax.ShapeDtypeStruct((N, FC), jnp.float32)

    @functools.partial(
        pl.kernel,
        out_type=[osh] * 4,
        mesh=mesh,
        scratch_types=[
            pltpu.VMEM_SHARED((N, FC), jnp.float32),
            pltpu.VMEM((SB, EB), jnp.int32),
            pltpu.VMEM((SB, EB), jnp.int32),
            pltpu.VMEM((SB, EB), jnp.float32),
            pltpu.VMEM((EB, FC), jnp.float32),
        ],
        compiler_params=pltpu.CompilerParams(use_tc_tiling_on_sc=False),
    )
    def k(h1a_h, h1b_h, h2a_h, h2b_h, src_h, dst_h, ew_h, z_h,
          o1a, o1b, o2a, o2b, acc, srcv, dstv, ewv, rows):
        c = lax.axis_index("c")
        s = lax.axis_index("s")

        def run_chunk(table, out):
            def super_iter(u, carry):
                pltpu.sync_copy(src_h.at[s, u], srcv)
                pltpu.sync_copy(dst_h.at[s, u], dstv)
                pltpu.sync_copy(ew_h.at[s, u], ewv)

                def batch(j, c2):
                    pltpu.sync_copy(table.at[srcv.at[j]], rows)
                    _scale_rows(rows, ewv, j, FC // LANES)
                    pltpu.sync_copy(rows, acc.at[dstv.at[j]], add=True)
                    return c2

                lax.fori_loop(0, SB, batch, 0)
                return carry

            lax.fori_loop(0, NSB, super_iter, 0)
            plsc.subcore_barrier()
            _writeout(acc, out, s)

        def run_branch(ta, tb, oa, ob):
            _zero_own(z_h, acc, s)
            plsc.subcore_barrier()
            run_chunk(ta, oa)
            _zero_own(z_h, acc, s)
            plsc.subcore_barrier()
            run_chunk(tb, ob)

        @pl.when(c == 0)
        def _():
            run_branch(h1a_h, h1b_h, o1a, o1b)

        @pl.when(c == 1)
        def _():
            run_branch(h2a_h, h2b_h, o2a, o2b)

    return k(h1a, h1b, h2a, h2b, srcr, dstr, ewr, zeros)


def _prop2(zp, srcr, dstr, ewr, zeros):
    mesh = plsc.VectorSubcoreMesh(
        core_axis_name="c", subcore_axis_name="s", num_cores=NC, num_subcores=NS
    )
    osh = jax.ShapeDtypeStruct((N, ZP), jnp.float32)

    @functools.partial(
        pl.kernel,
        out_type=[osh] * 2,
        mesh=mesh,
        scratch_types=[
            pltpu.VMEM_SHARED((N, ZP), jnp.float32),
            pltpu.VMEM((NB2, EB), jnp.int32),
            pltpu.VMEM((NB2, EB), jnp.int32),
            pltpu.VMEM((NB2, EB), jnp.float32),
            pltpu.VMEM((EB, ZP), jnp.float32),
        ],
        compiler_params=pltpu.CompilerParams(use_tc_tiling_on_sc=False),
    )
    def k(zp_h, src_h, dst_h, ew_h, z_h, o1, o2,
          acc, srcv, dstv, ewv, rows):
        c = lax.axis_index("c")
        s = lax.axis_index("s")
        w = s * NC + c
        pltpu.sync_copy(src_h.at[w], srcv)
        pltpu.sync_copy(dst_h.at[w], dstv)
        pltpu.sync_copy(ew_h.at[w], ewv)
        _zero_own(z_h, acc, s)
        plsc.subcore_barrier()

        def run(out):
            def batch(j, c2):
                pltpu.sync_copy(zp_h.at[srcv.at[j]], rows)
                _scale_rows(rows, ewv, j, ZP // LANES)
                pltpu.sync_copy(rows, acc.at[dstv.at[j]], add=True)
                return c2

            lax.fori_loop(0, NB2, batch, 0)
            plsc.subcore_barrier()
            _writeout(acc, out, s)

        @pl.when(c == 0)
        def _():
            run(o1)

        @pl.when(c == 1)
        def _():
            run(o2)

    return k(zp, srcr, dstr, ewr, zeros)


def _mid_body(a1a, a1b, a2a, a2b, b1a, b1b, b2a, b2b,
              wc1a, wc1b, wc2a, wc2b, o_ref):
    z = jnp.dot(jax.nn.relu(a1a[...] + b1a[...]), wc1a[...],
                preferred_element_type=jnp.float32)
    z = z + jnp.dot(jax.nn.relu(a1b[...] + b1b[...]), wc1b[...],
                    preferred_element_type=jnp.float32)
    z = z + jnp.dot(jax.nn.relu(a2a[...] + b2a[...]), wc2a[...],
                    preferred_element_type=jnp.float32)
    z = z + jnp.dot(jax.nn.relu(a2b[...] + b2b[...]), wc2b[...],
                    preferred_element_type=jnp.float32)
    o_ref[...] = z


def _mid(aggs, b1s, wcs):
    bm = 1000
    aspec = pl.BlockSpec((bm, FC), lambda i: (i, 0))
    bspec = pl.BlockSpec((1, FC), lambda i: (0, 0))
    wspec = pl.BlockSpec((FC, ZP), lambda i: (0, 0))
    return pl.pallas_call(
        _mid_body,
        grid=(N // bm,),
        in_specs=[aspec] * 4 + [bspec] * 4 + [wspec] * 4,
        out_specs=pl.BlockSpec((bm, ZP), lambda i: (i, 0)),
        out_shape=jax.ShapeDtypeStruct((N, ZP), jnp.float32),
    )(*aggs, *b1s, *wcs)


def _final_body(q0_ref, q1_ref, bc_ref, o_ref):
    t = q0_ref[...] + q1_ref[...]
    a0 = t[:, 0:1] + bc_ref[0, 0]
    a1 = t[:, 1:2] + bc_ref[0, 1]
    b0 = t[:, 2:3] + bc_ref[0, 2]
    b1 = t[:, 3:4] + bc_ref[0, 3]
    m1 = jnp.maximum(a0, a1)
    e0 = jnp.exp(a0 - m1)
    e1 = jnp.exp(a1 - m1)
    p10 = e0 / (e0 + e1)
    p11 = e1 / (e0 + e1)
    m2 = jnp.maximum(b0, b1)
    f0 = jnp.exp(b0 - m2)
    f1 = jnp.exp(b1 - m2)
    p20 = f0 / (f0 + f1)
    p21 = f1 / (f0 + f1)
    v0 = jnp.maximum(p10, p20)
    v1 = jnp.maximum(p11, p21)
    tot = v0 + v1
    o_ref[...] = jnp.concatenate([v0 / tot, v1 / tot], axis=1)


def _final(q0, q1, bc):
    bm = 1000
    return pl.pallas_call(
        _final_body,
        grid=(N // bm,),
        in_specs=[
            pl.BlockSpec((bm, ZP), lambda i: (i, 0)),
            pl.BlockSpec((bm, ZP), lambda i: (i, 0)),
            pl.BlockSpec((1, 4), lambda i: (0, 0)),
        ],
        out_specs=pl.BlockSpec((bm, 2), lambda i: (i, 0)),
        out_shape=jax.ShapeDtypeStruct((N, 2), jnp.float32),
    )(q0, q1, bc)


def _pad_cols(w, width):
    return jnp.pad(w, ((0, 0), (0, width - w.shape[1])))


def kernel(x, edge_index, edge_weight, W1_e1, b1_e1, Wc_e1, bc_e1,
           W1_e2, b1_e2, Wc_e2, bc_e2):
    src = edge_index[0].astype(jnp.int32)
    dst = edge_index[1].astype(jnp.int32)
    ew = edge_weight.astype(jnp.float32)

    w1a = W1_e1[:, :FC]
    w1b = _pad_cols(W1_e1[:, FC:], FC)
    w2a = W1_e2[:, :FC]
    w2b = _pad_cols(W1_e2[:, FC:], FC)
    b1a = b1_e1[:FC].reshape(1, FC)
    b1b = jnp.pad(b1_e1[FC:], (0, 2 * FC - HF)).reshape(1, FC)
    b2a = b1_e2[:FC].reshape(1, FC)
    b2b = jnp.pad(b1_e2[FC:], (0, 2 * FC - HF)).reshape(1, FC)
    wc1a = jnp.zeros((FC, ZP), jnp.float32).at[:, 0:2].set(Wc_e1[:FC])
    wc1b = jnp.zeros((FC, ZP), jnp.float32).at[:HF - FC, 0:2].set(Wc_e1[FC:])
    wc2a = jnp.zeros((FC, ZP), jnp.float32).at[:, 2:4].set(Wc_e2[:FC])
    wc2b = jnp.zeros((FC, ZP), jnp.float32).at[:HF - FC, 2:4].set(Wc_e2[FC:])
    bc = jnp.concatenate([bc_e1, bc_e2]).reshape(1, 4)

    src1 = src.reshape(NS, NSB, SB, EB)
    dst1 = dst.reshape(NS, NSB, SB, EB)
    ew1 = ew.reshape(NS, NSB, SB, EB)
    src2 = src.reshape(NC * NS, NB2, EB)
    dst2 = dst.reshape(NC * NS, NB2, EB)
    ew2 = ew.reshape(NC * NS, NB2, EB)
    zeros1 = jnp.zeros((WCHUNK, FC), jnp.float32)
    zeros2 = jnp.zeros((WCHUNK, ZP), jnp.float32)

    h1a, h1b, h2a, h2b = _matmuls(x, w1a, w1b, w2a, w2b)
    aggs = _prop1(h1a, h1b, h2a, h2b, src1, dst1, ew1, zeros1)
    zpk = _mid(aggs, (b1a, b1b, b2a, b2b), (wc1a, wc1b, wc2a, wc2b))
    q0, q1 = _prop2(zpk, src2, dst2, ew2, zeros2)
    return _final(q0, q1, bc)


# speedup vs baseline: 9.4791x; 1.8366x over previous
import functools

import jax
import jax.numpy as jnp
from jax import lax
from jax.experimental import pallas as pl
from jax.experimental.pallas import tpu as pltpu
from jax.experimental.pallas import tpu_sc as plsc

N = 10000
E = 320000
HF = 200
FC = 112
ZP = 16
NC = 2
NS = 16
EB = 80
NRING = 5
SB = 50
NSB = E // NS // (SB * EB)
NB2 = E // (NC * NS) // EB
WCHUNK = 1000
NWTILES = N // WCHUNK
LANES = 16


def _mm_body(x_ref, w1a, w1b, w2a, w2b, o1a, o1b, o2a, o2b):
    xa = x_ref[:, :768]
    xb = x_ref[:, 768:]
    o1a[...] = jnp.dot(xa, w1a[...], preferred_element_type=jnp.float32)
    o1b[...] = jnp.dot(xa, w1b[...], preferred_element_type=jnp.float32)
    o2a[...] = jnp.dot(xb, w2a[...], preferred_element_type=jnp.float32)
    o2b[...] = jnp.dot(xb, w2b[...], preferred_element_type=jnp.float32)


def _matmuls(x, w1a, w1b, w2a, w2b):
    bm = 1000
    wspec = pl.BlockSpec((768, FC), lambda i: (0, 0))
    ospec = pl.BlockSpec((bm, FC), lambda i: (i, 0))
    osh = jax.ShapeDtypeStruct((N, FC), jnp.float32)
    return pl.pallas_call(
        _mm_body,
        grid=(N // bm,),
        in_specs=[pl.BlockSpec((bm, 1536), lambda i: (i, 0))] + [wspec] * 4,
        out_specs=[ospec] * 4,
        out_shape=[osh] * 4,
    )(x, w1a, w1b, w2a, w2b)


def _zero_own(z_hbm, acc, s):
    @pl.when(s < NWTILES)
    def _():
        base = pl.multiple_of(s * WCHUNK, 8)
        pltpu.sync_copy(z_hbm, acc.at[pl.ds(base, WCHUNK)])


def _writeout(acc, out, s):
    @pl.when(s < NWTILES)
    def _():
        base = pl.multiple_of(s * WCHUNK, 8)
        pltpu.sync_copy(acc.at[pl.ds(base, WCHUNK)], out.at[pl.ds(base, WCHUNK)])


def _scale_rows(rowbuf, ewv, j, nvec):
    def scale_group(g, c2):
        wv = ewv[j, pl.ds(g * LANES, LANES)]
        for l in range(LANES):
            w = wv[l]
            for k in range(nvec):
                rowbuf[g * LANES + l, pl.ds(k * LANES, LANES)] = (
                    rowbuf[g * LANES + l, pl.ds(k * LANES, LANES)] * w
                )
        return c2

    lax.fori_loop(0, EB // LANES, scale_group, 0)


def _ring_pass(table, acc, srcv, dstv, ewv, rows, gsems, ssems, nvec, nb):
    nsteps = nb // NRING

    def gather(j, b):
        return pltpu.async_copy(table.at[srcv.at[j]], rows.at[b], gsems[b])

    def scatter(j, b):
        return pltpu.async_copy(rows.at[b], acc.at[dstv.at[j]], ssems[b],
                                add=True)

    def wait_gather(j, b):
        pltpu.make_async_copy(table.at[srcv.at[j]], rows.at[b],
                              gsems[b]).wait()

    def wait_scatter(j, b):
        pltpu.make_async_copy(rows.at[b], acc.at[dstv.at[j]],
                              ssems[b]).wait()

    gather(0, 0)
    gather(1, 1)

    def step(s, carry):
        j0 = s * NRING
        for l in range(NRING):
            j = j0 + l
            b = l
            bp = (l + 2) % NRING
            wait_gather(j, b)
            _scale_rows(rows.at[b], ewv, j, nvec)
            scatter(j, b)
            if l < 3:
                @pl.when(s >= 1)
                def _():
                    wait_scatter(j - 3, bp)
                gather(j + 2, bp)
            else:
                wait_scatter(j - 3, bp)

                @pl.when(s < nsteps - 1)
                def _():
                    gather(j + 2, bp)
        return carry

    lax.fori_loop(0, nsteps, step, 0)
    wait_scatter(nb - 3, 2)
    wait_scatter(nb - 2, 3)
    wait_scatter(nb - 1, 4)


def _prop1(h1a, h1b, h2a, h2b, srcr, dstr, ewr, zeros):
    mesh = plsc.VectorSubcoreMesh(
        core_axis_name="c", subcore_axis_name="s", num_cores=NC, num_subcores=NS
    )
    osh = jax.ShapeDtypeStruct((N, FC), jnp.float32)

    @functools.partial(
        pl.kernel,
        out_type=[osh] * 4,
        mesh=mesh,
        scratch_types=[
            pltpu.VMEM_SHARED((N, FC), jnp.float32),
            pltpu.VMEM((SB, EB), jnp.int32),
            pltpu.VMEM((SB, EB), jnp.int32),
            pltpu.VMEM((SB, EB), jnp.float32),
            pltpu.VMEM((NRING, EB, FC), jnp.float32),
        ] + [pltpu.SemaphoreType.DMA] * (2 * NRING),
        compiler_params=pltpu.CompilerParams(use_tc_tiling_on_sc=False),
    )
    def k(h1a_h, h1b_h, h2a_h, h2b_h, src_h, dst_h, ew_h, z_h,
          o1a, o1b, o2a, o2b, acc, srcv, dstv, ewv, rows, *sems):
        gsems = sems[:NRING]
        ssems = sems[NRING:]
        c = lax.axis_index("c")
        s = lax.axis_index("s")

        def run_chunk(table, out):
            def super_iter(u, carry):
                pltpu.sync_copy(src_h.at[s, u], srcv)
                pltpu.sync_copy(dst_h.at[s, u], dstv)
                pltpu.sync_copy(ew_h.at[s, u], ewv)
                _ring_pass(table, acc, srcv, dstv, ewv, rows,
                           gsems, ssems, FC // LANES, SB)
                return carry

            lax.fori_loop(0, NSB, super_iter, 0)
            plsc.subcore_barrier()
            _writeout(acc, out, s)

        def run_branch(ta, tb, oa, ob):
            _zero_own(z_h, acc, s)
            plsc.subcore_barrier()
            run_chunk(ta, oa)
            _zero_own(z_h, acc, s)
            plsc.subcore_barrier()
            run_chunk(tb, ob)

        @pl.when(c == 0)
        def _():
            run_branch(h1a_h, h1b_h, o1a, o1b)

        @pl.when(c == 1)
        def _():
            run_branch(h2a_h, h2b_h, o2a, o2b)

    return k(h1a, h1b, h2a, h2b, srcr, dstr, ewr, zeros)


def _prop2(zp, srcr, dstr, ewr, zeros):
    mesh = plsc.VectorSubcoreMesh(
        core_axis_name="c", subcore_axis_name="s", num_cores=NC, num_subcores=NS
    )
    osh = jax.ShapeDtypeStruct((N, ZP), jnp.float32)

    @functools.partial(
        pl.kernel,
        out_type=[osh] * 2,
        mesh=mesh,
        scratch_types=[
            pltpu.VMEM_SHARED((N, ZP), jnp.float32),
            pltpu.VMEM((NB2, EB), jnp.int32),
            pltpu.VMEM((NB2, EB), jnp.int32),
            pltpu.VMEM((NB2, EB), jnp.float32),
            pltpu.VMEM((NRING, EB, ZP), jnp.float32),
        ] + [pltpu.SemaphoreType.DMA] * (2 * NRING),
        compiler_params=pltpu.CompilerParams(use_tc_tiling_on_sc=False),
    )
    def k(zp_h, src_h, dst_h, ew_h, z_h, o1, o2,
          acc, srcv, dstv, ewv, rows, *sems):
        gsems = sems[:NRING]
        ssems = sems[NRING:]
        c = lax.axis_index("c")
        s = lax.axis_index("s")
        w = s * NC + c
        pltpu.sync_copy(src_h.at[w], srcv)
        pltpu.sync_copy(dst_h.at[w], dstv)
        pltpu.sync_copy(ew_h.at[w], ewv)
        _zero_own(z_h, acc, s)
        plsc.subcore_barrier()

        def run(out):
            _ring_pass(zp_h, acc, srcv, dstv, ewv, rows,
                       gsems, ssems, ZP // LANES, NB2)
            plsc.subcore_barrier()
            _writeout(acc, out, s)

        @pl.when(c == 0)
        def _():
            run(o1)

        @pl.when(c == 1)
        def _():
            run(o2)

    return k(zp, srcr, dstr, ewr, zeros)


def _mid_body(a1a, a1b, a2a, a2b, b1a, b1b, b2a, b2b,
              wc1a, wc1b, wc2a, wc2b, o_ref):
    z = jnp.dot(jax.nn.relu(a1a[...] + b1a[...]), wc1a[...],
                preferred_element_type=jnp.float32)
    z = z + jnp.dot(jax.nn.relu(a1b[...] + b1b[...]), wc1b[...],
                    preferred_element_type=jnp.float32)
    z = z + jnp.dot(jax.nn.relu(a2a[...] + b2a[...]), wc2a[...],
                    preferred_element_type=jnp.float32)
    z = z + jnp.dot(jax.nn.relu(a2b[...] + b2b[...]), wc2b[...],
                    preferred_element_type=jnp.float32)
    o_ref[...] = z


def _mid(aggs, b1s, wcs):
    bm = 1000
    aspec = pl.BlockSpec((bm, FC), lambda i: (i, 0))
    bspec = pl.BlockSpec((1, FC), lambda i: (0, 0))
    wspec = pl.BlockSpec((FC, ZP), lambda i: (0, 0))
    return pl.pallas_call(
        _mid_body,
        grid=(N // bm,),
        in_specs=[aspec] * 4 + [bspec] * 4 + [wspec] * 4,
        out_specs=pl.BlockSpec((bm, ZP), lambda i: (i, 0)),
        out_shape=jax.ShapeDtypeStruct((N, ZP), jnp.float32),
    )(*aggs, *b1s, *wcs)


def _final_body(q0_ref, q1_ref, bc_ref, o_ref):
    t = q0_ref[...] + q1_ref[...]
    a0 = t[:, 0:1] + bc_ref[0, 0]
    a1 = t[:, 1:2] + bc_ref[0, 1]
    b0 = t[:, 2:3] + bc_ref[0, 2]
    b1 = t[:, 3:4] + bc_ref[0, 3]
    m1 = jnp.maximum(a0, a1)
    e0 = jnp.exp(a0 - m1)
    e1 = jnp.exp(a1 - m1)
    p10 = e0 / (e0 + e1)
    p11 = e1 / (e0 + e1)
    m2 = jnp.maximum(b0, b1)
    f0 = jnp.exp(b0 - m2)
    f1 = jnp.exp(b1 - m2)
    p20 = f0 / (f0 + f1)
    p21 = f1 / (f0 + f1)
    v0 = jnp.maximum(p10, p20)
    v1 = jnp.maximum(p11, p21)
    tot = v0 + v1
    o_ref[...] = jnp.concatenate([v0 / tot, v1 / tot], axis=1)


def _final(q0, q1, bc):
    bm = 1000
    return pl.pallas_call(
        _final_body,
        grid=(N // bm,),
        in_specs=[
            pl.BlockSpec((bm, ZP), lambda i: (i, 0)),
            pl.BlockSpec((bm, ZP), lambda i: (i, 0)),
            pl.BlockSpec((1, 4), lambda i: (0, 0)),
        ],
        out_specs=pl.BlockSpec((bm, 2), lambda i: (i, 0)),
        out_shape=jax.ShapeDtypeStruct((N, 2), jnp.float32),
    )(q0, q1, bc)


def _pad_cols(w, width):
    return jnp.pad(w, ((0, 0), (0, width - w.shape[1])))


def kernel(x, edge_index, edge_weight, W1_e1, b1_e1, Wc_e1, bc_e1,
           W1_e2, b1_e2, Wc_e2, bc_e2):
    src = edge_index[0].astype(jnp.int32)
    dst = edge_index[1].astype(jnp.int32)
    ew = edge_weight.astype(jnp.float32)

    w1a = W1_e1[:, :FC]
    w1b = _pad_cols(W1_e1[:, FC:], FC)
    w2a = W1_e2[:, :FC]
    w2b = _pad_cols(W1_e2[:, FC:], FC)
    b1a = b1_e1[:FC].reshape(1, FC)
    b1b = jnp.pad(b1_e1[FC:], (0, 2 * FC - HF)).reshape(1, FC)
    b2a = b1_e2[:FC].reshape(1, FC)
    b2b = jnp.pad(b1_e2[FC:], (0, 2 * FC - HF)).reshape(1, FC)
    wc1a = jnp.zeros((FC, ZP), jnp.float32).at[:, 0:2].set(Wc_e1[:FC])
    wc1b = jnp.zeros((FC, ZP), jnp.float32).at[:HF - FC, 0:2].set(Wc_e1[FC:])
    wc2a = jnp.zeros((FC, ZP), jnp.float32).at[:, 2:4].set(Wc_e2[:FC])
    wc2b = jnp.zeros((FC, ZP), jnp.float32).at[:HF - FC, 2:4].set(Wc_e2[FC:])
    bc = jnp.concatenate([bc_e1, bc_e2]).reshape(1, 4)

    src1 = src.reshape(NS, NSB, SB, EB)
    dst1 = dst.reshape(NS, NSB, SB, EB)
    ew1 = ew.reshape(NS, NSB, SB, EB)
    src2 = src.reshape(NC * NS, NB2, EB)
    dst2 = dst.reshape(NC * NS, NB2, EB)
    ew2 = ew.reshape(NC * NS, NB2, EB)
    zeros1 = jnp.zeros((WCHUNK, FC), jnp.float32)
    zeros2 = jnp.zeros((WCHUNK, ZP), jnp.float32)

    h1a, h1b, h2a, h2b = _matmuls(x, w1a, w1b, w2a, w2b)
    aggs = _prop1(h1a, h1b, h2a, h2b, src1, dst1, ew1, zeros1)
    zpk = _mid(aggs, (b1a, b1b, b2a, b2b), (wc1a, wc1b, wc2a, wc2b))
    q0, q1 = _prop2(zpk, src2, dst2, ew2, zeros2)
    return _final(q0, q1, bc)


# speedup vs baseline: 10.1800x; 1.0739x over previous
import functools

import jax
import jax.numpy as jnp
from jax import lax
from jax.experimental import pallas as pl
from jax.experimental.pallas import tpu as pltpu
from jax.experimental.pallas import tpu_sc as plsc

N = 10000
E = 320000
HF = 200
FC = 112
ZP = 16
NC = 2
NS = 16
EB = 80
NRING = 5
SB = 50
NSB = E // NS // (SB * EB)
NB2 = E // (NC * NS) // EB
WCHUNK = 1000
NWTILES = N // WCHUNK
LANES = 16


def _mm_body(x_ref, w1a, w1b, w2a, w2b, o1a, o1b, o2a, o2b):
    xa = x_ref[:, :768]
    xb = x_ref[:, 768:]
    o1a[...] = jnp.dot(xa, w1a[...], preferred_element_type=jnp.float32)
    o1b[...] = jnp.dot(xa, w1b[...], preferred_element_type=jnp.float32)
    o2a[...] = jnp.dot(xb, w2a[...], preferred_element_type=jnp.float32)
    o2b[...] = jnp.dot(xb, w2b[...], preferred_element_type=jnp.float32)


def _matmuls(x, w1a, w1b, w2a, w2b):
    bm = 1000
    wspec = pl.BlockSpec((768, FC), lambda i: (0, 0))
    ospec = pl.BlockSpec((bm, FC), lambda i: (i, 0))
    osh = jax.ShapeDtypeStruct((N, FC), jnp.float32)
    return pl.pallas_call(
        _mm_body,
        grid=(N // bm,),
        in_specs=[pl.BlockSpec((bm, 1536), lambda i: (i, 0))] + [wspec] * 4,
        out_specs=[ospec] * 4,
        out_shape=[osh] * 4,
    )(x, w1a, w1b, w2a, w2b)


def _zero_own(z_hbm, acc, s):
    @pl.when(s < NWTILES)
    def _():
        base = pl.multiple_of(s * WCHUNK, 8)
        pltpu.sync_copy(z_hbm, acc.at[pl.ds(base, WCHUNK)])


def _writeout(acc, out, s):
    @pl.when(s < NWTILES)
    def _():
        base = pl.multiple_of(s * WCHUNK, 8)
        pltpu.sync_copy(acc.at[pl.ds(base, WCHUNK)], out.at[pl.ds(base, WCHUNK)])


def _scale_rows(rowbuf, ewv, j, nvec):
    def scale_group(g, c2):
        wv = ewv[j, pl.ds(g * LANES, LANES)]
        for l in range(LANES):
            w = wv[l]
            for k in range(nvec):
                rowbuf[g * LANES + l, pl.ds(k * LANES, LANES)] = (
                    rowbuf[g * LANES + l, pl.ds(k * LANES, LANES)] * w
                )
        return c2

    lax.fori_loop(0, EB // LANES, scale_group, 0)


def _ring_pass(table, acc, srcv, dstv, ewv, rows, gsems, ssems, nvec, nb):
    nsteps = nb // NRING

    def gather(j, b):
        return pltpu.async_copy(table.at[srcv.at[j]], rows.at[b], gsems[b])

    def scatter(j, b):
        return pltpu.async_copy(rows.at[b], acc.at[dstv.at[j]], ssems[b],
                                add=True)

    def wait_gather(j, b):
        pltpu.make_async_copy(table.at[srcv.at[j]], rows.at[b],
                              gsems[b]).wait()

    def wait_scatter(j, b):
        pltpu.make_async_copy(rows.at[b], acc.at[dstv.at[j]],
                              ssems[b]).wait()

    gather(0, 0)
    gather(1, 1)

    def step(s, carry):
        j0 = s * NRING
        for l in range(NRING):
            j = j0 + l
            b = l
            bp = (l + 2) % NRING
            if l < 3:
                @pl.when(s >= 1)
                def _():
                    wait_scatter(j - 3, bp)
                gather(j + 2, bp)
            else:
                wait_scatter(j - 3, bp)

                @pl.when(s < nsteps - 1)
                def _():
                    gather(j + 2, bp)
            wait_gather(j, b)
            _scale_rows(rows.at[b], ewv, j, nvec)
            scatter(j, b)
        return carry

    lax.fori_loop(0, nsteps, step, 0)
    wait_scatter(nb - 3, 2)
    wait_scatter(nb - 2, 3)
    wait_scatter(nb - 1, 4)


def _prop1(h1a, h1b, h2a, h2b, srcr, dstr, ewr, zeros):
    mesh = plsc.VectorSubcoreMesh(
        core_axis_name="c", subcore_axis_name="s", num_cores=NC, num_subcores=NS
    )
    osh = jax.ShapeDtypeStruct((N, FC), jnp.float32)

    @functools.partial(
        pl.kernel,
        out_type=[osh] * 4,
        mesh=mesh,
        scratch_types=[
            pltpu.VMEM_SHARED((N, FC), jnp.float32),
            pltpu.VMEM((SB, EB), jnp.int32),
            pltpu.VMEM((SB, EB), jnp.int32),
            pltpu.VMEM((SB, EB), jnp.float32),
            pltpu.VMEM((NRING, EB, FC), jnp.float32),
        ] + [pltpu.SemaphoreType.DMA] * (2 * NRING),
        compiler_params=pltpu.CompilerParams(use_tc_tiling_on_sc=False),
    )
    def k(h1a_h, h1b_h, h2a_h, h2b_h, src_h, dst_h, ew_h, z_h,
          o1a, o1b, o2a, o2b, acc, srcv, dstv, ewv, rows, *sems):
        gsems = sems[:NRING]
        ssems = sems[NRING:]
        c = lax.axis_index("c")
        s = lax.axis_index("s")

        def run_chunk(table, out):
            def super_iter(u, carry):
                pltpu.sync_copy(src_h.at[s, u], srcv)
                pltpu.sync_copy(dst_h.at[s, u], dstv)
                pltpu.sync_copy(ew_h.at[s, u], ewv)
                _ring_pass(table, acc, srcv, dstv, ewv, rows,
                           gsems, ssems, FC // LANES, SB)
                return carry

            lax.fori_loop(0, NSB, super_iter, 0)
            plsc.subcore_barrier()
            _writeout(acc, out, s)

        def run_branch(ta, tb, oa, ob):
            _zero_own(z_h, acc, s)
            plsc.subcore_barrier()
            run_chunk(ta, oa)
            _zero_own(z_h, acc, s)
            plsc.subcore_barrier()
            run_chunk(tb, ob)

        @pl.when(c == 0)
        def _():
            run_branch(h1a_h, h1b_h, o1a, o1b)

        @pl.when(c == 1)
        def _():
            run_branch(h2a_h, h2b_h, o2a, o2b)

    return k(h1a, h1b, h2a, h2b, srcr, dstr, ewr, zeros)


def _prop2(zp, srcr, dstr, ewr, zeros):
    mesh = plsc.VectorSubcoreMesh(
        core_axis_name="c", subcore_axis_name="s", num_cores=NC, num_subcores=NS
    )
    osh = jax.ShapeDtypeStruct((N, ZP), jnp.float32)

    @functools.partial(
        pl.kernel,
        out_type=[osh] * 2,
        mesh=mesh,
        scratch_types=[
            pltpu.VMEM_SHARED((N, ZP), jnp.float32),
            pltpu.VMEM((NB2, EB), jnp.int32),
            pltpu.VMEM((NB2, EB), jnp.int32),
            pltpu.VMEM((NB2, EB), jnp.float32),
            pltpu.VMEM((NRING, EB, ZP), jnp.float32),
        ] + [pltpu.SemaphoreType.DMA] * (2 * NRING),
        compiler_params=pltpu.CompilerParams(use_tc_tiling_on_sc=False),
    )
    def k(zp_h, src_h, dst_h, ew_h, z_h, o1, o2,
          acc, srcv, dstv, ewv, rows, *sems):
        gsems = sems[:NRING]
        ssems = sems[NRING:]
        c = lax.axis_index("c")
        s = lax.axis_index("s")
        w = s * NC + c
        pltpu.sync_copy(src_h.at[w], srcv)
        pltpu.sync_copy(dst_h.at[w], dstv)
        pltpu.sync_copy(ew_h.at[w], ewv)
        _zero_own(z_h, acc, s)
        plsc.subcore_barrier()

        def run(out):
            _ring_pass(zp_h, acc, srcv, dstv, ewv, rows,
                       gsems, ssems, ZP // LANES, NB2)
            plsc.subcore_barrier()
            _writeout(acc, out, s)

        @pl.when(c == 0)
        def _():
            run(o1)

        @pl.when(c == 1)
        def _():
            run(o2)

    return k(zp, srcr, dstr, ewr, zeros)


def _mid_body(a1a, a1b, a2a, a2b, b1a, b1b, b2a, b2b,
              wc1a, wc1b, wc2a, wc2b, o_ref):
    z = jnp.dot(jax.nn.relu(a1a[...] + b1a[...]), wc1a[...],
                preferred_element_type=jnp.float32)
    z = z + jnp.dot(jax.nn.relu(a1b[...] + b1b[...]), wc1b[...],
                    preferred_element_type=jnp.float32)
    z = z + jnp.dot(jax.nn.relu(a2a[...] + b2a[...]), wc2a[...],
                    preferred_element_type=jnp.float32)
    z = z + jnp.dot(jax.nn.relu(a2b[...] + b2b[...]), wc2b[...],
                    preferred_element_type=jnp.float32)
    o_ref[...] = z


def _mid(aggs, b1s, wcs):
    bm = 1000
    aspec = pl.BlockSpec((bm, FC), lambda i: (i, 0))
    bspec = pl.BlockSpec((1, FC), lambda i: (0, 0))
    wspec = pl.BlockSpec((FC, ZP), lambda i: (0, 0))
    return pl.pallas_call(
        _mid_body,
        grid=(N // bm,),
        in_specs=[aspec] * 4 + [bspec] * 4 + [wspec] * 4,
        out_specs=pl.BlockSpec((bm, ZP), lambda i: (i, 0)),
        out_shape=jax.ShapeDtypeStruct((N, ZP), jnp.float32),
    )(*aggs, *b1s, *wcs)


def _final_body(q0_ref, q1_ref, bc_ref, o_ref):
    t = q0_ref[...] + q1_ref[...]
    a0 = t[:, 0:1] + bc_ref[0, 0]
    a1 = t[:, 1:2] + bc_ref[0, 1]
    b0 = t[:, 2:3] + bc_ref[0, 2]
    b1 = t[:, 3:4] + bc_ref[0, 3]
    m1 = jnp.maximum(a0, a1)
    e0 = jnp.exp(a0 - m1)
    e1 = jnp.exp(a1 - m1)
    p10 = e0 / (e0 + e1)
    p11 = e1 / (e0 + e1)
    m2 = jnp.maximum(b0, b1)
    f0 = jnp.exp(b0 - m2)
    f1 = jnp.exp(b1 - m2)
    p20 = f0 / (f0 + f1)
    p21 = f1 / (f0 + f1)
    v0 = jnp.maximum(p10, p20)
    v1 = jnp.maximum(p11, p21)
    tot = v0 + v1
    o_ref[...] = jnp.concatenate([v0 / tot, v1 / tot], axis=1)


def _final(q0, q1, bc):
    bm = 1000
    return pl.pallas_call(
        _final_body,
        grid=(N // bm,),
        in_specs=[
            pl.BlockSpec((bm, ZP), lambda i: (i, 0)),
            pl.BlockSpec((bm, ZP), lambda i: (i, 0)),
            pl.BlockSpec((1, 4), lambda i: (0, 0)),
        ],
        out_specs=pl.BlockSpec((bm, 2), lambda i: (i, 0)),
        out_shape=jax.ShapeDtypeStruct((N, 2), jnp.float32),
    )(q0, q1, bc)


def _pad_cols(w, width):
    return jnp.pad(w, ((0, 0), (0, width - w.shape[1])))


def kernel(x, edge_index, edge_weight, W1_e1, b1_e1, Wc_e1, bc_e1,
           W1_e2, b1_e2, Wc_e2, bc_e2):
    src = edge_index[0].astype(jnp.int32)
    dst = edge_index[1].astype(jnp.int32)
    ew = edge_weight.astype(jnp.float32)

    w1a = W1_e1[:, :FC]
    w1b = _pad_cols(W1_e1[:, FC:], FC)
    w2a = W1_e2[:, :FC]
    w2b = _pad_cols(W1_e2[:, FC:], FC)
    b1a = b1_e1[:FC].reshape(1, FC)
    b1b = jnp.pad(b1_e1[FC:], (0, 2 * FC - HF)).reshape(1, FC)
    b2a = b1_e2[:FC].reshape(1, FC)
    b2b = jnp.pad(b1_e2[FC:], (0, 2 * FC - HF)).reshape(1, FC)
    wc1a = jnp.zeros((FC, ZP), jnp.float32).at[:, 0:2].set(Wc_e1[:FC])
    wc1b = jnp.zeros((FC, ZP), jnp.float32).at[:HF - FC, 0:2].set(Wc_e1[FC:])
    wc2a = jnp.zeros((FC, ZP), jnp.float32).at[:, 2:4].set(Wc_e2[:FC])
    wc2b = jnp.zeros((FC, ZP), jnp.float32).at[:HF - FC, 2:4].set(Wc_e2[FC:])
    bc = jnp.concatenate([bc_e1, bc_e2]).reshape(1, 4)

    src1 = src.reshape(NS, NSB, SB, EB)
    dst1 = dst.reshape(NS, NSB, SB, EB)
    ew1 = ew.reshape(NS, NSB, SB, EB)
    src2 = src.reshape(NC * NS, NB2, EB)
    dst2 = dst.reshape(NC * NS, NB2, EB)
    ew2 = ew.reshape(NC * NS, NB2, EB)
    zeros1 = jnp.zeros((WCHUNK, FC), jnp.float32)
    zeros2 = jnp.zeros((WCHUNK, ZP), jnp.float32)

    h1a, h1b, h2a, h2b = _matmuls(x, w1a, w1b, w2a, w2b)
    aggs = _prop1(h1a, h1b, h2a, h2b, src1, dst1, ew1, zeros1)
    zpk = _mid(aggs, (b1a, b1b, b2a, b2b), (wc1a, wc1b, wc2a, wc2b))
    q0, q1 = _prop2(zpk, src2, dst2, ew2, zeros2)
    return _final(q0, q1, bc)
